# baseline (speedup 1.0000x reference)
import functools

import jax
import jax.numpy as jnp
import numpy as np
from jax import lax
from jax.experimental import pallas as pl
from jax.experimental.pallas import tpu as pltpu
from jax.experimental.pallas import tpu_sc as plsc

N_NODES = 10000
N_EDGES = 160000
MUL_IN = 16
MUL_OUT = 16
DIM_EDGE_EMB = 64
HIDDEN = 16
WNUM = MUL_IN * MUL_OUT
NORM = 1.0 / np.sqrt(MUL_IN)

NC = 2
NS = 16
NW = NC * NS

CH = 40
ACC_ROWS = 10240
ZROWS = ACC_ROWS // NS
NBUF = 4

CPT_A = 64
CPT_B = 61
E_A = CPT_A * NW * CH
E_B = CPT_B * NW * CH
BLK_E = 1280
GRID_A = E_A // BLK_E
GRID_B = E_B // BLK_E


@functools.cache
def _sc_kernels(cpt):
    mesh = plsc.VectorSubcoreMesh(core_axis_name="c", subcore_axis_name="s",
                                  num_cores=NC, num_subcores=NS)
    n_edges = cpt * NW * CH

    @functools.partial(
        pl.kernel,
        out_type=jax.ShapeDtypeStruct((n_edges, 128), jnp.float32),
        mesh=mesh,
        scratch_types=[
            pltpu.VMEM((cpt, CH), jnp.int32),
            pltpu.VMEM((NBUF, CH, 128), jnp.float32),
            pltpu.SemaphoreType.DMA,
            pltpu.SemaphoreType.DMA,
        ],
    )
    def gather_k(table_hbm, src_hbm, out_hbm, idx_v, rows_v, gsem, osem):
        wid = lax.axis_index("c") * NS + lax.axis_index("s")
        c0 = wid * cpt
        pltpu.sync_copy(src_hbm.at[wid], idx_v)

        for b in range(NBUF):
            pltpu.async_copy(table_hbm.at[idx_v.at[b]], rows_v.at[b], gsem)

        def body(j, _):
            buf = rows_v.at[lax.rem(j, NBUF)]
            pltpu.make_async_copy(table_hbm.at[idx_v.at[j]], buf, gsem).wait()
            ocp = pltpu.async_copy(buf, out_hbm.at[pl.ds((c0 + j) * CH, CH)], osem)

            @pl.when(j + NBUF < cpt)
            def _prefetch():
                ocp.wait()
                pltpu.async_copy(table_hbm.at[idx_v.at[j + NBUF]], buf, gsem)

            return _

        lax.fori_loop(0, cpt, body, None)
        for b in range(NBUF):
            pltpu.make_async_copy(rows_v.at[b],
                                  out_hbm.at[pl.ds(c0 * CH, CH)], osem).wait()

    @functools.partial(
        pl.kernel,
        out_type=jax.ShapeDtypeStruct((NC, ACC_ROWS, 128), jnp.float32),
        mesh=mesh,
        scratch_types=[
            pltpu.VMEM((cpt, CH), jnp.int32),
            pltpu.VMEM((NBUF, CH, 128), jnp.float32),
            pltpu.VMEM_SHARED((ACC_ROWS, 128), jnp.float32),
            pltpu.SemaphoreType.DMA,
            pltpu.SemaphoreType.DMA,
        ],
    )
    def scatter_k(feat_hbm, dst_hbm, zeros_hbm, part_hbm, idx_v, feat_v, acc_sh,
                  lsem, ssem):
        cid = lax.axis_index("c")
        sid = lax.axis_index("s")
        r0 = sid * ZROWS
        pltpu.sync_copy(zeros_hbm, acc_sh.at[pl.ds(r0, ZROWS)])
        wid = cid * NS + sid
        c0 = wid * cpt
        pltpu.sync_copy(dst_hbm.at[wid], idx_v)
        plsc.subcore_barrier()

        for b in range(NBUF):
            pltpu.async_copy(feat_hbm.at[pl.ds((c0 + b) * CH, CH)],
                             feat_v.at[b], lsem)

        def body(j, _):
            buf = feat_v.at[lax.rem(j, NBUF)]
            pltpu.make_async_copy(feat_hbm.at[pl.ds((c0 + j) * CH, CH)],
                                  buf, lsem).wait()
            scp = pltpu.async_copy(buf, acc_sh.at[idx_v.at[j]], ssem, add=True)

            @pl.when(j + NBUF < cpt)
            def _prefetch():
                scp.wait()
                pltpu.async_copy(feat_hbm.at[pl.ds((c0 + j + NBUF) * CH, CH)],
                                 buf, lsem)

            return _

        lax.fori_loop(0, cpt, body, None)
        for b in range(NBUF):
            pltpu.make_async_copy(feat_v.at[b], acc_sh.at[idx_v.at[b]], ssem).wait()
        plsc.subcore_barrier()
        pltpu.sync_copy(acc_sh.at[pl.ds(r0, ZROWS)],
                        part_hbm.at[cid, pl.ds(r0, ZROWS)])

    return gather_k, scatter_k


def _compute_body(y_ref, embt_ref, attrt_ref, w1_ref, b1_ref, w2_ref, b2_ref,
                  r_ref, s_ref, p_ref, out_ref):
    x1 = y_ref[:, :MUL_IN]
    h = lax.dot_general(embt_ref[...], w1_ref[...], (((0,), (0,)), ((), ())),
                        preferred_element_type=jnp.float32)
    h = h + b1_ref[...]
    h = h * jax.nn.sigmoid(h)
    wt = jnp.dot(h, w2_ref[...], preferred_element_type=jnp.float32) + b2_ref[...]
    x1r = jnp.dot(x1, r_ref[...], preferred_element_type=jnp.float32)
    attr_col = lax.transpose(attrt_ref[...], (1, 0))
    ef = jnp.dot(wt * x1r, s_ref[...], preferred_element_type=jnp.float32)
    ef = ef * (attr_col * NORM)
    out_ref[...] = jnp.dot(ef, p_ref[...], preferred_element_type=jnp.float32)


def _make_compute(grid, base_blk, n_edges):
    return pl.pallas_call(
        _compute_body,
        grid=(grid,),
        in_specs=[
            pl.BlockSpec((BLK_E, 128), lambda i: (i, 0)),
            pl.BlockSpec((DIM_EDGE_EMB, BLK_E), lambda i: (0, base_blk + i)),
            pl.BlockSpec((1, BLK_E), lambda i: (0, base_blk + i)),
            pl.BlockSpec((DIM_EDGE_EMB, HIDDEN), lambda i: (0, 0)),
            pl.BlockSpec((1, HIDDEN), lambda i: (0, 0)),
            pl.BlockSpec((HIDDEN, WNUM), lambda i: (0, 0)),
            pl.BlockSpec((1, WNUM), lambda i: (0, 0)),
            pl.BlockSpec((MUL_IN, WNUM), lambda i: (0, 0)),
            pl.BlockSpec((WNUM, MUL_OUT), lambda i: (0, 0)),
            pl.BlockSpec((MUL_OUT, 128), lambda i: (0, 0)),
        ],
        out_specs=pl.BlockSpec((BLK_E, 128), lambda i: (i, 0)),
        out_shape=jax.ShapeDtypeStruct((n_edges, 128), jnp.float32),
    )


_compute_a = _make_compute(GRID_A, 0, E_A)
_compute_b = _make_compute(GRID_B, GRID_A, E_B)


BLK_N = 1000
GRID_N = N_NODES // BLK_N


def _combine_body(pa_ref, pb_ref, out_ref):
    out_ref[...] = (pa_ref[0, :, :MUL_OUT] + pa_ref[1, :, :MUL_OUT]
                    + pb_ref[0, :, :MUL_OUT] + pb_ref[1, :, :MUL_OUT])


_combine_k = pl.pallas_call(
    _combine_body,
    grid=(GRID_N,),
    in_specs=[pl.BlockSpec((NC, BLK_N, 128), lambda i: (0, i, 0)),
              pl.BlockSpec((NC, BLK_N, 128), lambda i: (0, i, 0))],
    out_specs=pl.BlockSpec((BLK_N, MUL_OUT), lambda i: (i, 0)),
    out_shape=jax.ShapeDtypeStruct((N_NODES, MUL_OUT), jnp.float32),
)


def kernel(node_features, edge_src, edge_dst, edge_attr, edge_embedding,
           W1, b1, W2, b2):
    gather_a, scatter_a = _sc_kernels(CPT_A)
    gather_b, scatter_b = _sc_kernels(CPT_B)
    src = edge_src.astype(jnp.int32)
    dst = edge_dst.astype(jnp.int32)
    src_a = src[:E_A].reshape(NW, CPT_A, CH)
    src_b = src[E_A:].reshape(NW, CPT_B, CH)
    dst_a = dst[:E_A].reshape(NW, CPT_A, CH)
    dst_b = dst[E_A:].reshape(NW, CPT_B, CH)
    table = jnp.tile(node_features, (1, 128 // MUL_IN))

    y_a = gather_a(table, src_a)
    y_b = gather_b(table, src_b)

    k256 = jnp.arange(WNUM)
    R = (k256[None, :] // MUL_OUT == jnp.arange(MUL_IN)[:, None]).astype(jnp.float32)
    S = (k256[:, None] % MUL_OUT == jnp.arange(MUL_OUT)[None, :]).astype(jnp.float32)
    P = (jnp.arange(128)[None, :] == jnp.arange(MUL_OUT)[:, None]).astype(jnp.float32)

    embt = jnp.swapaxes(edge_embedding, 0, 1)
    attrt = jnp.swapaxes(edge_attr, 0, 1)
    consts = (W1, b1.reshape(1, -1), W2, b2.reshape(1, -1), R, S, P)
    efw_a = _compute_a(y_a, embt, attrt, *consts)
    efw_b = _compute_b(y_b, embt, attrt, *consts)

    zeros = jnp.zeros((ZROWS, 128), jnp.float32)
    part_a = scatter_a(efw_a, dst_a, zeros)
    part_b = scatter_b(efw_b, dst_b, zeros)
    return _combine_k(part_a, part_b)


# speedup vs baseline: 1.1098x; 1.1098x over previous
import functools

import jax
import jax.numpy as jnp
import numpy as np
from jax import lax
from jax.experimental import pallas as pl
from jax.experimental.pallas import tpu as pltpu
from jax.experimental.pallas import tpu_sc as plsc

N_NODES = 10000
N_EDGES = 160000
MUL_IN = 16
MUL_OUT = 16
DIM_EDGE_EMB = 64
HIDDEN = 16
WNUM = MUL_IN * MUL_OUT
NORM = 1.0 / np.sqrt(MUL_IN)

NC = 2
NS = 16
NW = NC * NS

CH = 40
NCHUNK = N_EDGES // CH
CPT = NCHUNK // NW
CPT_SC = (NCHUNK // NC) // NS
ACC_ROWS = 10240
ZROWS = ACC_ROWS // NS
NBUF = 4


@functools.cache
def _sc_kernels():
    mesh = plsc.VectorSubcoreMesh(core_axis_name="c", subcore_axis_name="s",
                                  num_cores=NC, num_subcores=NS)

    @functools.partial(
        pl.kernel,
        out_type=jax.ShapeDtypeStruct((N_EDGES, 128), jnp.float32),
        mesh=mesh,
        scratch_types=[
            pltpu.VMEM((CPT, CH), jnp.int32),
            pltpu.VMEM((NBUF, CH, 128), jnp.float32),
            pltpu.SemaphoreType.DMA,
            pltpu.SemaphoreType.DMA,
        ],
    )
    def gather_k(table_hbm, src_hbm, out_hbm, idx_v, rows_v, gsem, osem):
        wid = lax.axis_index("c") * NS + lax.axis_index("s")
        c0 = wid * CPT
        pltpu.sync_copy(src_hbm.at[wid], idx_v)

        for b in range(NBUF):
            pltpu.async_copy(table_hbm.at[idx_v.at[b]], rows_v.at[b], gsem)

        def body(j, _):
            buf = rows_v.at[lax.rem(j, NBUF)]
            pltpu.make_async_copy(table_hbm.at[idx_v.at[j]], buf, gsem).wait()
            ocp = pltpu.async_copy(buf, out_hbm.at[pl.ds((c0 + j) * CH, CH)], osem)

            @pl.when(j + NBUF < CPT)
            def _prefetch():
                ocp.wait()
                pltpu.async_copy(table_hbm.at[idx_v.at[j + NBUF]], buf, gsem)

            return _

        lax.fori_loop(0, CPT, body, None)
        for b in range(NBUF):
            pltpu.make_async_copy(rows_v.at[b],
                                  out_hbm.at[pl.ds(c0 * CH, CH)], osem).wait()

    @functools.partial(
        pl.kernel,
        out_type=jax.ShapeDtypeStruct((NC, ACC_ROWS, 128), jnp.float32),
        mesh=mesh,
        scratch_types=[
            pltpu.VMEM((CPT_SC, CH), jnp.int32),
            pltpu.VMEM((NBUF, CH, 128), jnp.float32),
            pltpu.VMEM_SHARED((ACC_ROWS, 128), jnp.float32),
            pltpu.SemaphoreType.DMA,
            pltpu.SemaphoreType.DMA,
        ],
    )
    def scatter_k(feat_hbm, dst_hbm, zeros_hbm, part_hbm, idx_v, feat_v, acc_sh,
                  lsem, ssem):
        cid = lax.axis_index("c")
        sid = lax.axis_index("s")
        r0 = sid * ZROWS
        pltpu.sync_copy(zeros_hbm, acc_sh.at[pl.ds(r0, ZROWS)])
        wid = cid * NS + sid
        c0 = wid * CPT_SC
        pltpu.sync_copy(dst_hbm.at[wid], idx_v)
        plsc.subcore_barrier()

        for b in range(NBUF):
            pltpu.async_copy(feat_hbm.at[pl.ds((c0 + b) * CH, CH)],
                             feat_v.at[b], lsem)

        def body(j, _):
            buf = feat_v.at[lax.rem(j, NBUF)]
            pltpu.make_async_copy(feat_hbm.at[pl.ds((c0 + j) * CH, CH)],
                                  buf, lsem).wait()
            scp = pltpu.async_copy(buf, acc_sh.at[idx_v.at[j]], ssem, add=True)

            @pl.when(j + NBUF < CPT_SC)
            def _prefetch():
                scp.wait()
                pltpu.async_copy(feat_hbm.at[pl.ds((c0 + j + NBUF) * CH, CH)],
                                 buf, lsem)

            return _

        lax.fori_loop(0, CPT_SC, body, None)
        for b in range(NBUF):
            pltpu.make_async_copy(feat_v.at[b], acc_sh.at[idx_v.at[b]], ssem).wait()
        plsc.subcore_barrier()
        pltpu.sync_copy(acc_sh.at[pl.ds(r0, ZROWS)],
                        part_hbm.at[cid, pl.ds(r0, ZROWS)])

    return gather_k, scatter_k


BLK_E = 3200
GRID_E = N_EDGES // BLK_E


def _compute_body(y_ref, embt_ref, attrt_ref, w1_ref, b1_ref, w2_ref, b2_ref,
                  r_ref, sp_ref, out_ref):
    x1 = y_ref[:, :MUL_IN]
    h = lax.dot_general(embt_ref[...], w1_ref[...], (((0,), (0,)), ((), ())),
                        preferred_element_type=jnp.float32)
    h = h + b1_ref[...]
    h = h * jax.nn.sigmoid(h)
    wt = jnp.dot(h, w2_ref[...], preferred_element_type=jnp.float32) + b2_ref[...]
    x1r = jnp.dot(x1, r_ref[...], preferred_element_type=jnp.float32)
    attr_col = lax.transpose(attrt_ref[...], (1, 0))
    prod = wt * x1r * (attr_col * NORM)
    out_ref[...] = jnp.dot(prod, sp_ref[...], preferred_element_type=jnp.float32)


_compute_k = pl.pallas_call(
    _compute_body,
    grid=(GRID_E,),
    in_specs=[
        pl.BlockSpec((BLK_E, 128), lambda i: (i, 0)),
        pl.BlockSpec((DIM_EDGE_EMB, BLK_E), lambda i: (0, i)),
        pl.BlockSpec((1, BLK_E), lambda i: (0, i)),
        pl.BlockSpec((DIM_EDGE_EMB, HIDDEN), lambda i: (0, 0)),
        pl.BlockSpec((1, HIDDEN), lambda i: (0, 0)),
        pl.BlockSpec((HIDDEN, WNUM), lambda i: (0, 0)),
        pl.BlockSpec((1, WNUM), lambda i: (0, 0)),
        pl.BlockSpec((MUL_IN, WNUM), lambda i: (0, 0)),
        pl.BlockSpec((WNUM, 128), lambda i: (0, 0)),
    ],
    out_specs=pl.BlockSpec((BLK_E, 128), lambda i: (i, 0)),
    out_shape=jax.ShapeDtypeStruct((N_EDGES, 128), jnp.float32),
)


BLK_N = 1000
GRID_N = N_NODES // BLK_N


def _combine_body(p_ref, out_ref):
    out_ref[...] = p_ref[0, :, :MUL_OUT] + p_ref[1, :, :MUL_OUT]


_combine_k = pl.pallas_call(
    _combine_body,
    grid=(GRID_N,),
    in_specs=[pl.BlockSpec((NC, BLK_N, 128), lambda i: (0, i, 0))],
    out_specs=pl.BlockSpec((BLK_N, MUL_OUT), lambda i: (i, 0)),
    out_shape=jax.ShapeDtypeStruct((N_NODES, MUL_OUT), jnp.float32),
)


def kernel(node_features, edge_src, edge_dst, edge_attr, edge_embedding,
           W1, b1, W2, b2):
    gather_k, scatter_k = _sc_kernels()
    src3 = edge_src.astype(jnp.int32).reshape(NW, CPT, CH)
    dst3 = edge_dst.astype(jnp.int32).reshape(NW, CPT_SC, CH)
    table = jnp.tile(node_features, (1, 128 // MUL_IN))

    y = gather_k(table, src3)

    k256 = jnp.arange(WNUM)
    l128 = jnp.arange(128)
    R = (k256[None, :] // MUL_OUT == jnp.arange(MUL_IN)[:, None]).astype(jnp.float32)
    SP = ((k256[:, None] % MUL_OUT == l128[None, :])
          & (l128[None, :] < MUL_OUT)).astype(jnp.float32)

    embt = jnp.swapaxes(edge_embedding, 0, 1)
    attrt = jnp.swapaxes(edge_attr, 0, 1)
    efw = _compute_k(y, embt, attrt, W1, b1.reshape(1, -1), W2, b2.reshape(1, -1),
                     R, SP)

    zeros = jnp.zeros((ZROWS, 128), jnp.float32)
    partials = scatter_k(efw, dst3, zeros)
    return _combine_k(partials)


# speedup vs baseline: 1.1192x; 1.0085x over previous
import functools

import jax
import jax.numpy as jnp
import numpy as np
from jax import lax
from jax.experimental import pallas as pl
from jax.experimental.pallas import tpu as pltpu
from jax.experimental.pallas import tpu_sc as plsc

N_NODES = 10000
N_EDGES = 160000
MUL_IN = 16
MUL_OUT = 16
DIM_EDGE_EMB = 64
HIDDEN = 16
WNUM = MUL_IN * MUL_OUT
NORM = 1.0 / np.sqrt(MUL_IN)

NC = 2
NS = 16
NW = NC * NS

CH = 40
NCHUNK = N_EDGES // CH
CPT = NCHUNK // NW
CPT_SC = (NCHUNK // NC) // NS
ACC_ROWS = 10240
ZROWS = ACC_ROWS // NS
NBUF_G = 8
NBUF_S = 4


@functools.cache
def _sc_kernels():
    mesh = plsc.VectorSubcoreMesh(core_axis_name="c", subcore_axis_name="s",
                                  num_cores=NC, num_subcores=NS)

    @functools.partial(
        pl.kernel,
        out_type=jax.ShapeDtypeStruct((N_EDGES, 128), jnp.float32),
        mesh=mesh,
        scratch_types=[
            pltpu.VMEM((CPT, CH), jnp.int32),
            pltpu.VMEM((NBUF_G, CH, 128), jnp.float32),
            pltpu.SemaphoreType.DMA,
            pltpu.SemaphoreType.DMA,
        ],
    )
    def gather_k(table_hbm, src_hbm, out_hbm, idx_v, rows_v, gsem, osem):
        wid = lax.axis_index("c") * NS + lax.axis_index("s")
        c0 = wid * CPT
        pltpu.sync_copy(src_hbm.at[wid], idx_v)

        for b in range(NBUF_G):
            pltpu.async_copy(table_hbm.at[idx_v.at[b]], rows_v.at[b], gsem)

        def body(j, _):
            buf = rows_v.at[lax.rem(j, NBUF_G)]
            pltpu.make_async_copy(table_hbm.at[idx_v.at[j]], buf, gsem).wait()
            ocp = pltpu.async_copy(buf, out_hbm.at[pl.ds((c0 + j) * CH, CH)], osem)

            @pl.when(j + NBUF_G < CPT)
            def _prefetch():
                ocp.wait()
                pltpu.async_copy(table_hbm.at[idx_v.at[j + NBUF_G]], buf, gsem)

            return _

        lax.fori_loop(0, CPT, body, None)
        for b in range(NBUF_G):
            pltpu.make_async_copy(rows_v.at[b],
                                  out_hbm.at[pl.ds(c0 * CH, CH)], osem).wait()

    @functools.partial(
        pl.kernel,
        out_type=jax.ShapeDtypeStruct((NC, ACC_ROWS, 128), jnp.float32),
        mesh=mesh,
        scratch_types=[
            pltpu.VMEM((CPT_SC, CH), jnp.int32),
            pltpu.VMEM((NBUF_S, CH, 128), jnp.float32),
            pltpu.VMEM_SHARED((ACC_ROWS, 128), jnp.float32),
            pltpu.SemaphoreType.DMA,
            pltpu.SemaphoreType.DMA,
        ],
    )
    def scatter_k(feat_hbm, dst_hbm, zeros_hbm, part_hbm, idx_v, feat_v, acc_sh,
                  lsem, ssem):
        cid = lax.axis_index("c")
        sid = lax.axis_index("s")
        r0 = sid * ZROWS
        pltpu.sync_copy(zeros_hbm, acc_sh.at[pl.ds(r0, ZROWS)])
        wid = cid * NS + sid
        c0 = wid * CPT_SC
        pltpu.sync_copy(dst_hbm.at[wid], idx_v)
        plsc.subcore_barrier()

        for b in range(NBUF_S):
            pltpu.async_copy(feat_hbm.at[pl.ds((c0 + b) * CH, CH)],
                             feat_v.at[b], lsem)

        def body(j, _):
            buf = feat_v.at[lax.rem(j, NBUF_S)]
            pltpu.make_async_copy(feat_hbm.at[pl.ds((c0 + j) * CH, CH)],
                                  buf, lsem).wait()
            scp = pltpu.async_copy(buf, acc_sh.at[idx_v.at[j]], ssem, add=True)

            @pl.when(j + NBUF_S < CPT_SC)
            def _prefetch():
                scp.wait()
                pltpu.async_copy(feat_hbm.at[pl.ds((c0 + j + NBUF_S) * CH, CH)],
                                 buf, lsem)

            return _

        lax.fori_loop(0, CPT_SC, body, None)
        for b in range(NBUF_S):
            pltpu.make_async_copy(feat_v.at[b], acc_sh.at[idx_v.at[b]], ssem).wait()
        plsc.subcore_barrier()
        pltpu.sync_copy(acc_sh.at[pl.ds(r0, ZROWS)],
                        part_hbm.at[cid, pl.ds(r0, ZROWS)])

    return gather_k, scatter_k


BLK_E = 3200
GRID_E = N_EDGES // BLK_E


def _compute_body(y_ref, embt_ref, attrt_ref, w1_ref, b1_ref, w2_ref, b2_ref,
                  r_ref, sp_ref, out_ref):
    x1 = y_ref[:, :MUL_IN]
    h = lax.dot_general(embt_ref[...], w1_ref[...], (((0,), (0,)), ((), ())),
                        preferred_element_type=jnp.float32)
    h = h + b1_ref[...]
    h = h * jax.nn.sigmoid(h)
    wt = jnp.dot(h, w2_ref[...], preferred_element_type=jnp.float32) + b2_ref[...]
    x1r = jnp.dot(x1, r_ref[...], preferred_element_type=jnp.float32)
    attr_col = lax.transpose(attrt_ref[...], (1, 0))
    prod = wt * x1r * (attr_col * NORM)
    out_ref[...] = jnp.dot(prod, sp_ref[...], preferred_element_type=jnp.float32)


_compute_k = pl.pallas_call(
    _compute_body,
    grid=(GRID_E,),
    in_specs=[
        pl.BlockSpec((BLK_E, 128), lambda i: (i, 0)),
        pl.BlockSpec((DIM_EDGE_EMB, BLK_E), lambda i: (0, i)),
        pl.BlockSpec((1, BLK_E), lambda i: (0, i)),
        pl.BlockSpec((DIM_EDGE_EMB, HIDDEN), lambda i: (0, 0)),
        pl.BlockSpec((1, HIDDEN), lambda i: (0, 0)),
        pl.BlockSpec((HIDDEN, WNUM), lambda i: (0, 0)),
        pl.BlockSpec((1, WNUM), lambda i: (0, 0)),
        pl.BlockSpec((MUL_IN, WNUM), lambda i: (0, 0)),
        pl.BlockSpec((WNUM, 128), lambda i: (0, 0)),
    ],
    out_specs=pl.BlockSpec((BLK_E, 128), lambda i: (i, 0)),
    out_shape=jax.ShapeDtypeStruct((N_EDGES, 128), jnp.float32),
)


BLK_N = 1000
GRID_N = N_NODES // BLK_N


def _combine_body(p_ref, out_ref):
    out_ref[...] = p_ref[0, :, :MUL_OUT] + p_ref[1, :, :MUL_OUT]


_combine_k = pl.pallas_call(
    _combine_body,
    grid=(GRID_N,),
    in_specs=[pl.BlockSpec((NC, BLK_N, 128), lambda i: (0, i, 0))],
    out_specs=pl.BlockSpec((BLK_N, MUL_OUT), lambda i: (i, 0)),
    out_shape=jax.ShapeDtypeStruct((N_NODES, MUL_OUT), jnp.float32),
)


def kernel(node_features, edge_src, edge_dst, edge_attr, edge_embedding,
           W1, b1, W2, b2):
    gather_k, scatter_k = _sc_kernels()
    src3 = edge_src.astype(jnp.int32).reshape(NW, CPT, CH)
    dst3 = edge_dst.astype(jnp.int32).reshape(NW, CPT_SC, CH)
    table = jnp.tile(node_features, (1, 128 // MUL_IN))

    y = gather_k(table, src3)

    k256 = jnp.arange(WNUM)
    l128 = jnp.arange(128)
    R = (k256[None, :] // MUL_OUT == jnp.arange(MUL_IN)[:, None]).astype(jnp.float32)
    SP = ((k256[:, None] % MUL_OUT == l128[None, :])
          & (l128[None, :] < MUL_OUT)).astype(jnp.float32)

    embt = jnp.swapaxes(edge_embedding, 0, 1)
    attrt = jnp.swapaxes(edge_attr, 0, 1)
    efw = _compute_k(y, embt, attrt, W1, b1.reshape(1, -1), W2, b2.reshape(1, -1),
                     R, SP)

    zeros = jnp.zeros((ZROWS, 128), jnp.float32)
    partials = scatter_k(efw, dst3, zeros)
    return _combine_k(partials)
